# Initial kernel scaffold; baseline (speedup 1.0000x reference)
import jax
import jax.numpy as jnp
from jax.experimental import pallas as pl


def kernel(x, edge_index, W1, b1, W2, b2):
    raise NotImplementedError("write your pallas kernel here")


# speedup vs baseline: 4.6697x; 4.6697x over previous
import jax
import jax.numpy as jnp
from jax import lax
from jax.experimental import pallas as pl
from jax.experimental.pallas import tpu as pltpu
from jax.experimental.pallas import tpu_sc as plsc

_N = 10000
_E = 320000
_NPAD = 10240
_NC = 2
_NS = 16
_NW = _NC * _NS
_CHUNK = 80
_STRIPE = _NPAD // _NS
_WIN = 2304
_NWIN = 5
_NPAD2 = _NWIN * _WIN
_ACC = 2432
_AZ = _ACC // _NS
_AW = _WIN // _NS


def _sc_mesh():
    return plsc.VectorSubcoreMesh(core_axis_name="c", subcore_axis_name="s")


def _make_sc_degree():
    n_chunks = _E // _NW // _CHUNK

    def body(dst3, ones_hbm, zinit, out, didx, didx1, ones_v, acc):
        c = lax.axis_index("c")
        t = lax.axis_index("s")
        w = c * _NS + t
        pltpu.sync_copy(zinit, acc.at[pl.ds(t * _STRIPE, _STRIPE)])
        pltpu.sync_copy(dst3.at[w], didx)
        pltpu.sync_copy(ones_hbm, ones_v)
        plsc.subcore_barrier()

        def step(j, carry):
            base = j * _CHUNK
            for k in range(5):
                didx1[pl.ds(k * 16, 16)] = didx[0, pl.ds(base + k * 16, 16)]
            pltpu.sync_copy(ones_v, acc.at[didx1], add=True)
            return carry

        lax.fori_loop(0, n_chunks, step, 0)
        plsc.subcore_barrier()
        pltpu.sync_copy(acc.at[pl.ds(t * _STRIPE, _STRIPE)],
                        out.at[pl.ds(c * _NPAD + t * _STRIPE, _STRIPE)])

    return pl.kernel(
        body,
        out_type=jax.ShapeDtypeStruct((_NC * _NPAD,), jnp.float32),
        mesh=_sc_mesh(),
        scratch_types=[
            pltpu.VMEM((1, _E // _NW), jnp.int32),
            pltpu.VMEM((_CHUNK,), jnp.int32),
            pltpu.VMEM((_CHUNK,), jnp.float32),
            pltpu.VMEM_SHARED((_NPAD,), jnp.float32),
        ],
    )


def _make_sc_agg(edge_split):
    if edge_split:
        ept = _E // _NC // _NS
    else:
        ept = _E // _NS
    n_groups = ept // 16
    n_pairs = ept // 160
    tail = ept - n_pairs * 160

    def body(src3, dst3, table, out,
             sidx, didx2, s1a, d1a, s1b, d1b,
             rows0, rows1, zbuf, sem0, sem1, acc):
        c = lax.axis_index("c")
        t = lax.axis_index("s")
        if edge_split:
            w = c * _NS + t
        else:
            w = t

        def fillz(i, carry):
            r = i // 8
            k = i % 8
            zbuf[r, pl.ds(k * 16, 16)] = jnp.zeros((16,), jnp.float32)
            return carry

        lax.fori_loop(0, (_AZ // 4) * 8, fillz, 0)
        pltpu.sync_copy(src3.at[w], sidx)
        pltpu.sync_copy(dst3.at[w], didx2)
        if not edge_split:
            off = c * _N

            def addoff(g, carry):
                v = sidx[0, pl.ds(g * 16, 16)]
                sidx[0, pl.ds(g * 16, 16)] = v + off
                return carry

            lax.fori_loop(0, n_groups, addoff, 0)

        def copy80(src1d, base, dst80):
            for k in range(5):
                dst80[pl.ds(k * 16, 16)] = src1d[0, pl.ds(base + k * 16, 16)]

        def g_start(idx80, rows, sem):
            pltpu.async_copy(table.at[idx80], rows, sem)

        def g_wait(rows, sem):
            pltpu.make_async_copy(table.at[pl.ds(0, _CHUNK)], rows,
                                  sem).wait()

        def one_pass(p, dst3_again):
            lo = p * _WIN
            for q in range(4):
                pltpu.sync_copy(
                    zbuf, acc.at[pl.ds(t * _AZ + q * (_AZ // 4), _AZ // 4)])

            pltpu.sync_copy(dst3_again.at[w], didx2)

            def remap(g, carry):
                v = didx2[0, pl.ds(g * 16, 16)]
                ok = (v >= lo) & (v < lo + _WIN)
                junk = _WIN + (v & 0x7F)
                didx2[0, pl.ds(g * 16, 16)] = jnp.where(ok, v - lo, junk)
                return carry

            lax.fori_loop(0, n_groups, remap, 0)
            plsc.subcore_barrier()

            def pair(i, carry):
                base = i * 160
                copy80(sidx, base, s1a)
                copy80(didx2, base, d1a)
                g_start(s1a, rows0, sem0)
                copy80(sidx, base + 80, s1b)
                copy80(didx2, base + 80, d1b)
                g_start(s1b, rows1, sem1)
                g_wait(rows0, sem0)
                pltpu.sync_copy(rows0, acc.at[d1a], add=True)
                g_wait(rows1, sem1)
                pltpu.sync_copy(rows1, acc.at[d1b], add=True)
                return carry

            lax.fori_loop(0, n_pairs, pair, 0)
            if tail:
                base = n_pairs * 160
                copy80(sidx, base, s1a)
                copy80(didx2, base, d1a)
                g_start(s1a, rows0, sem0)
                g_wait(rows0, sem0)
                pltpu.sync_copy(rows0, acc.at[d1a], add=True)
            plsc.subcore_barrier()
            obase = c * _NPAD2 + p * _WIN + t * _AW
            pltpu.sync_copy(acc.at[pl.ds(t * _AW, _AW)],
                            out.at[pl.ds(obase, _AW)])
            plsc.subcore_barrier()

        def passes(p, carry):
            one_pass(p, dst3)
            return carry

        for p in range(_NWIN):
            one_pass(p, dst3)

    return pl.kernel(
        body,
        out_type=jax.ShapeDtypeStruct((_NC * _NPAD2, 128), jnp.float32),
        mesh=_sc_mesh(),
        scratch_types=[
            pltpu.VMEM((1, ept), jnp.int32),
            pltpu.VMEM((1, ept), jnp.int32),
            pltpu.VMEM((_CHUNK,), jnp.int32),
            pltpu.VMEM((_CHUNK,), jnp.int32),
            pltpu.VMEM((_CHUNK,), jnp.int32),
            pltpu.VMEM((_CHUNK,), jnp.int32),
            pltpu.VMEM((_CHUNK, 128), jnp.float32),
            pltpu.VMEM((_CHUNK, 128), jnp.float32),
            pltpu.VMEM((_AZ // 4, 128), jnp.float32),
            pltpu.SemaphoreType.DMA,
            pltpu.SemaphoreType.DMA,
            pltpu.VMEM_SHARED((_ACC, 128), jnp.float32),
        ],
    )


_BN = 1000


def _tc_deginv(degp):
    blk = 2048

    def body(d_ref, out_ref):
        deg = d_ref[0, :] + d_ref[1, :] + 1.0
        out_ref[...] = lax.rsqrt(deg)[:, None]

    return pl.pallas_call(
        body,
        grid=(_NPAD // blk,),
        in_specs=[pl.BlockSpec((2, blk), lambda i: (0, i))],
        out_specs=pl.BlockSpec((blk, 1), lambda i: (i, 0)),
        out_shape=jax.ShapeDtypeStruct((_NPAD, 1), jnp.float32),
    )(degp)


def _tc_stage1(dinv, x, W1s):
    def body(dv_ref, x_ref, w_ref, out_ref):
        h = jnp.dot(x_ref[...], w_ref[0],
                    preferred_element_type=jnp.float32)
        out_ref[0] = dv_ref[...] * h

    return pl.pallas_call(
        body,
        grid=(_N // _BN, 2),
        in_specs=[
            pl.BlockSpec((_BN, 1), lambda i, c: (i, 0)),
            pl.BlockSpec((_BN, 128), lambda i, c: (i, 0)),
            pl.BlockSpec((1, 128, 128), lambda i, c: (c, 0, 0)),
        ],
        out_specs=pl.BlockSpec((1, _BN, 128), lambda i, c: (c, i, 0)),
        out_shape=jax.ShapeDtypeStruct((2, _N, 128), jnp.float32),
    )(dinv, x, W1s)


def _tc_stage2(dinv, s1, g1, W2, b1r):
    def body(dv_ref, s_ref, g_ref, w_ref, b_ref, out_ref):
        dv = dv_ref[...]
        b = b_ref[...]
        w = w_ref[...]
        z0 = jnp.maximum(dv * (s_ref[0] + g_ref[0]) + b[0, :128], 0.0)
        z1 = jnp.maximum(dv * (s_ref[1] + g_ref[1]) + b[0, 128:], 0.0)
        h = (jnp.dot(z0, w[:128], preferred_element_type=jnp.float32)
             + jnp.dot(z1, w[128:], preferred_element_type=jnp.float32))
        out_ref[...] = dv * h

    return pl.pallas_call(
        body,
        grid=(_N // _BN,),
        in_specs=[
            pl.BlockSpec((_BN, 1), lambda i: (i, 0)),
            pl.BlockSpec((2, _BN, 128), lambda i: (0, i, 0)),
            pl.BlockSpec((2, _BN, 128), lambda i: (0, i, 0)),
            pl.BlockSpec((256, 128), lambda i: (0, 0)),
            pl.BlockSpec((1, 256), lambda i: (0, 0)),
        ],
        out_specs=pl.BlockSpec((_BN, 128), lambda i: (i, 0)),
        out_shape=jax.ShapeDtypeStruct((_N, 128), jnp.float32),
    )(dinv, s1, g1, W2, b1r)


def _tc_stage3(dinv, s2, g2, b2r):
    def body(dv_ref, s_ref, g_ref, b_ref, out_ref):
        tot = s_ref[0] + s_ref[1] + g_ref[...]
        out_ref[...] = jnp.maximum(dv_ref[...] * tot + b_ref[...], 0.0)

    return pl.pallas_call(
        body,
        grid=(_N // _BN,),
        in_specs=[
            pl.BlockSpec((_BN, 1), lambda i: (i, 0)),
            pl.BlockSpec((2, _BN, 128), lambda i: (0, i, 0)),
            pl.BlockSpec((_BN, 128), lambda i: (i, 0)),
            pl.BlockSpec((1, 128), lambda i: (0, 0)),
        ],
        out_specs=pl.BlockSpec((_BN, 128), lambda i: (i, 0)),
        out_shape=jax.ShapeDtypeStruct((_N, 128), jnp.float32),
    )(dinv, s2, g2, b2r)


def kernel(x, edge_index, W1, b1, W2, b2):
    src_all = edge_index[0].reshape(_NS, 1, _E // _NS)
    dst_all = edge_index[1].reshape(_NS, 1, _E // _NS)
    src_half = edge_index[0].reshape(_NW, 1, _E // _NW)
    dst_half = edge_index[1].reshape(_NW, 1, _E // _NW)
    deg_dst = edge_index[1].reshape(_NW, 1, _E // _NW)
    zeros1 = jnp.zeros((_STRIPE,), jnp.float32)
    ones1 = jnp.ones((_CHUNK,), jnp.float32)
    W1s = W1.reshape(128, 2, 128).transpose(1, 0, 2)

    degp = _make_sc_degree()(deg_dst, ones1, zeros1)
    dinv = _tc_deginv(degp.reshape(_NC, _NPAD))

    g1 = _tc_stage1(dinv, x, W1s)
    s1 = _make_sc_agg(edge_split=False)(
        src_all, dst_all, g1.reshape(2 * _N, 128))
    g2 = _tc_stage2(dinv, s1.reshape(_NC, _NPAD2, 128), g1, W2,
                    b1.reshape(1, 256))
    s2 = _make_sc_agg(edge_split=True)(
        src_half, dst_half, g2)
    return _tc_stage3(dinv, s2.reshape(_NC, _NPAD2, 128), g2,
                      b2.reshape(1, 128))


# speedup vs baseline: 7.4880x; 1.6035x over previous
import jax
import jax.numpy as jnp
from jax import lax
from jax.experimental import pallas as pl
from jax.experimental.pallas import tpu as pltpu
from jax.experimental.pallas import tpu_sc as plsc

_N = 10000
_E = 320000
_NPAD = 10240
_NC = 2
_NS = 16
_NW = _NC * _NS
_CHUNK = 80
_STRIPE = _NPAD // _NS
_WIN = 3456
_NWIN = 3
_NPAD2 = _NWIN * _WIN
_ACC = 3584
_AZ = _ACC // _NS
_AW = _WIN // _NS


def _sc_mesh():
    return plsc.VectorSubcoreMesh(core_axis_name="c", subcore_axis_name="s")


def _make_sc_degree():
    n_chunks = _E // _NW // _CHUNK

    def body(dst3, ones_hbm, zinit, out, didx, didx1, ones_v, acc):
        c = lax.axis_index("c")
        t = lax.axis_index("s")
        w = c * _NS + t
        pltpu.sync_copy(zinit, acc.at[pl.ds(t * _STRIPE, _STRIPE)])
        pltpu.sync_copy(dst3.at[w], didx)
        pltpu.sync_copy(ones_hbm, ones_v)
        plsc.subcore_barrier()

        def step(j, carry):
            base = j * _CHUNK
            for k in range(5):
                didx1[pl.ds(k * 16, 16)] = didx[0, pl.ds(base + k * 16, 16)]
            pltpu.sync_copy(ones_v, acc.at[didx1], add=True)
            return carry

        lax.fori_loop(0, n_chunks, step, 0)
        plsc.subcore_barrier()
        pltpu.sync_copy(acc.at[pl.ds(t * _STRIPE, _STRIPE)],
                        out.at[pl.ds(c * _NPAD + t * _STRIPE, _STRIPE)])

    return pl.kernel(
        body,
        out_type=jax.ShapeDtypeStruct((_NC * _NPAD,), jnp.float32),
        mesh=_sc_mesh(),
        scratch_types=[
            pltpu.VMEM((1, _E // _NW), jnp.int32),
            pltpu.VMEM((_CHUNK,), jnp.int32),
            pltpu.VMEM((_CHUNK,), jnp.float32),
            pltpu.VMEM_SHARED((_NPAD,), jnp.float32),
        ],
    )


def _make_sc_agg(edge_split):
    if edge_split:
        ept = _E // _NC // _NS
    else:
        ept = _E // _NS
    n_groups = ept // 16
    n_pairs = ept // 160
    tail = ept - n_pairs * 160

    def body(src3, dst3, table, out,
             sidx, didx2, s1a, d1a, s1b, d1b,
             rows0, rows1, zbuf, sem0, sem1, acc):
        c = lax.axis_index("c")
        t = lax.axis_index("s")
        if edge_split:
            w = c * _NS + t
        else:
            w = t

        def fillz(i, carry):
            r = i // 8
            k = i % 8
            zbuf[r, pl.ds(k * 16, 16)] = jnp.zeros((16,), jnp.float32)
            return carry

        lax.fori_loop(0, (_AZ // 4) * 8, fillz, 0)
        pltpu.sync_copy(src3.at[w], sidx)
        pltpu.sync_copy(dst3.at[w], didx2)
        if not edge_split:
            off = c * _N

            def addoff(g, carry):
                v = sidx[0, pl.ds(g * 16, 16)]
                sidx[0, pl.ds(g * 16, 16)] = v + off
                return carry

            lax.fori_loop(0, n_groups, addoff, 0)

        def copy80(src1d, base, dst80):
            for k in range(5):
                dst80[pl.ds(k * 16, 16)] = src1d[0, pl.ds(base + k * 16, 16)]

        def g_start(idx80, rows, sem):
            pltpu.async_copy(table.at[idx80], rows, sem)

        def g_wait(rows, sem):
            pltpu.make_async_copy(table.at[pl.ds(0, _CHUNK)], rows,
                                  sem).wait()

        def one_pass(p, dst3_again):
            lo = p * _WIN
            for q in range(4):
                pltpu.sync_copy(
                    zbuf, acc.at[pl.ds(t * _AZ + q * (_AZ // 4), _AZ // 4)])

            pltpu.sync_copy(dst3_again.at[w], didx2)

            def remap(g, carry):
                v = didx2[0, pl.ds(g * 16, 16)]
                ok = (v >= lo) & (v < lo + _WIN)
                junk = _WIN + (v & 0x7F)
                didx2[0, pl.ds(g * 16, 16)] = jnp.where(ok, v - lo, junk)
                return carry

            lax.fori_loop(0, n_groups, remap, 0)
            plsc.subcore_barrier()

            def pair(i, carry):
                base = i * 160
                copy80(sidx, base, s1a)
                copy80(didx2, base, d1a)
                g_start(s1a, rows0, sem0)
                copy80(sidx, base + 80, s1b)
                copy80(didx2, base + 80, d1b)
                g_start(s1b, rows1, sem1)
                g_wait(rows0, sem0)
                pltpu.sync_copy(rows0, acc.at[d1a], add=True)
                g_wait(rows1, sem1)
                pltpu.sync_copy(rows1, acc.at[d1b], add=True)
                return carry

            lax.fori_loop(0, n_pairs, pair, 0)
            if tail:
                base = n_pairs * 160
                copy80(sidx, base, s1a)
                copy80(didx2, base, d1a)
                g_start(s1a, rows0, sem0)
                g_wait(rows0, sem0)
                pltpu.sync_copy(rows0, acc.at[d1a], add=True)
            plsc.subcore_barrier()
            obase = c * _NPAD2 + p * _WIN + t * _AW
            pltpu.sync_copy(acc.at[pl.ds(t * _AW, _AW)],
                            out.at[pl.ds(obase, _AW)])
            plsc.subcore_barrier()

        def passes(p, carry):
            one_pass(p, dst3)
            return carry

        for p in range(_NWIN):
            one_pass(p, dst3)

    return pl.kernel(
        body,
        out_type=jax.ShapeDtypeStruct((_NC * _NPAD2, 128), jnp.float32),
        mesh=_sc_mesh(),
        scratch_types=[
            pltpu.VMEM((1, ept), jnp.int32),
            pltpu.VMEM((1, ept), jnp.int32),
            pltpu.VMEM((_CHUNK,), jnp.int32),
            pltpu.VMEM((_CHUNK,), jnp.int32),
            pltpu.VMEM((_CHUNK,), jnp.int32),
            pltpu.VMEM((_CHUNK,), jnp.int32),
            pltpu.VMEM((_CHUNK, 128), jnp.float32),
            pltpu.VMEM((_CHUNK, 128), jnp.float32),
            pltpu.VMEM((_AZ // 4, 128), jnp.float32),
            pltpu.SemaphoreType.DMA,
            pltpu.SemaphoreType.DMA,
            pltpu.VMEM_SHARED((_ACC, 128), jnp.float32),
        ],
    )


_BN = 1000


def _tc_deginv(degp):
    blk = 2048

    def body(d_ref, out_ref):
        deg = d_ref[0, :] + d_ref[1, :] + 1.0
        out_ref[...] = lax.rsqrt(deg)[:, None]

    return pl.pallas_call(
        body,
        grid=(_NPAD // blk,),
        in_specs=[pl.BlockSpec((2, blk), lambda i: (0, i))],
        out_specs=pl.BlockSpec((blk, 1), lambda i: (i, 0)),
        out_shape=jax.ShapeDtypeStruct((_NPAD, 1), jnp.float32),
    )(degp)


def _tc_stage1(dinv, x, W1s):
    def body(dv_ref, x_ref, w_ref, out_ref):
        h = jnp.dot(x_ref[...], w_ref[0],
                    preferred_element_type=jnp.float32)
        out_ref[0] = dv_ref[...] * h

    return pl.pallas_call(
        body,
        grid=(_N // _BN, 2),
        in_specs=[
            pl.BlockSpec((_BN, 1), lambda i, c: (i, 0)),
            pl.BlockSpec((_BN, 128), lambda i, c: (i, 0)),
            pl.BlockSpec((1, 128, 128), lambda i, c: (c, 0, 0)),
        ],
        out_specs=pl.BlockSpec((1, _BN, 128), lambda i, c: (c, i, 0)),
        out_shape=jax.ShapeDtypeStruct((2, _N, 128), jnp.float32),
    )(dinv, x, W1s)


def _tc_stage2(dinv, s1, g1, W2, b1r):
    def body(dv_ref, s_ref, g_ref, w_ref, b_ref, out_ref):
        dv = dv_ref[...]
        b = b_ref[...]
        w = w_ref[...]
        z0 = jnp.maximum(dv * (s_ref[0] + g_ref[0]) + b[0, :128], 0.0)
        z1 = jnp.maximum(dv * (s_ref[1] + g_ref[1]) + b[0, 128:], 0.0)
        h = (jnp.dot(z0, w[:128], preferred_element_type=jnp.float32)
             + jnp.dot(z1, w[128:], preferred_element_type=jnp.float32))
        out_ref[...] = dv * h

    return pl.pallas_call(
        body,
        grid=(_N // _BN,),
        in_specs=[
            pl.BlockSpec((_BN, 1), lambda i: (i, 0)),
            pl.BlockSpec((2, _BN, 128), lambda i: (0, i, 0)),
            pl.BlockSpec((2, _BN, 128), lambda i: (0, i, 0)),
            pl.BlockSpec((256, 128), lambda i: (0, 0)),
            pl.BlockSpec((1, 256), lambda i: (0, 0)),
        ],
        out_specs=pl.BlockSpec((_BN, 128), lambda i: (i, 0)),
        out_shape=jax.ShapeDtypeStruct((_N, 128), jnp.float32),
    )(dinv, s1, g1, W2, b1r)


def _tc_stage3(dinv, s2, g2, b2r):
    def body(dv_ref, s_ref, g_ref, b_ref, out_ref):
        tot = s_ref[0] + s_ref[1] + g_ref[...]
        out_ref[...] = jnp.maximum(dv_ref[...] * tot + b_ref[...], 0.0)

    return pl.pallas_call(
        body,
        grid=(_N // _BN,),
        in_specs=[
            pl.BlockSpec((_BN, 1), lambda i: (i, 0)),
            pl.BlockSpec((2, _BN, 128), lambda i: (0, i, 0)),
            pl.BlockSpec((_BN, 128), lambda i: (i, 0)),
            pl.BlockSpec((1, 128), lambda i: (0, 0)),
        ],
        out_specs=pl.BlockSpec((_BN, 128), lambda i: (i, 0)),
        out_shape=jax.ShapeDtypeStruct((_N, 128), jnp.float32),
    )(dinv, s2, g2, b2r)


def kernel(x, edge_index, W1, b1, W2, b2):
    src_all = edge_index[0].reshape(_NS, 1, _E // _NS)
    dst_all = edge_index[1].reshape(_NS, 1, _E // _NS)
    src_half = edge_index[0].reshape(_NW, 1, _E // _NW)
    dst_half = edge_index[1].reshape(_NW, 1, _E // _NW)
    deg_dst = edge_index[1].reshape(_NW, 1, _E // _NW)
    zeros1 = jnp.zeros((_STRIPE,), jnp.float32)
    ones1 = jnp.ones((_CHUNK,), jnp.float32)
    W1s = W1.reshape(128, 2, 128).transpose(1, 0, 2)

    degp = _make_sc_degree()(deg_dst, ones1, zeros1)
    dinv = _tc_deginv(degp.reshape(_NC, _NPAD))

    g1 = _tc_stage1(dinv, x, W1s)
    s1 = _make_sc_agg(edge_split=False)(
        src_all, dst_all, g1.reshape(2 * _N, 128))
    g2 = _tc_stage2(dinv, s1.reshape(_NC, _NPAD2, 128), g1, W2,
                    b1.reshape(1, 256))
    s2 = _make_sc_agg(edge_split=True)(
        src_half, dst_half, g2)
    return _tc_stage3(dinv, s2.reshape(_NC, _NPAD2, 128), g2,
                      b2.reshape(1, 128))


# speedup vs baseline: 11.4741x; 1.5323x over previous
import jax
import jax.numpy as jnp
from jax import lax
from jax.experimental import pallas as pl
from jax.experimental.pallas import tpu as pltpu
from jax.experimental.pallas import tpu_sc as plsc

_N = 10000
_E = 320000
_NPAD = 10240
_NC = 2
_NS = 16
_NW = _NC * _NS
_CHUNK = 80
_STRIPE = _NPAD // _NS
_WIN = 3456
_NWIN = 3
_NPAD2 = _NWIN * _WIN
_ACC = 3584
_AZ = _ACC // _NS
_AW = _WIN // _NS


def _sc_mesh():
    return plsc.VectorSubcoreMesh(core_axis_name="c", subcore_axis_name="s")


def _make_sc_degree():
    n_chunks = _E // _NW // _CHUNK

    def body(dst3, ones_hbm, zinit, out, didx, didx1, ones_v, acc):
        c = lax.axis_index("c")
        t = lax.axis_index("s")
        w = c * _NS + t
        pltpu.sync_copy(zinit, acc.at[pl.ds(t * _STRIPE, _STRIPE)])
        pltpu.sync_copy(dst3.at[w], didx)
        pltpu.sync_copy(ones_hbm, ones_v)
        plsc.subcore_barrier()

        def step(j, carry):
            base = j * _CHUNK
            for k in range(5):
                didx1[pl.ds(k * 16, 16)] = didx[0, pl.ds(base + k * 16, 16)]
            pltpu.sync_copy(ones_v, acc.at[didx1], add=True)
            return carry

        lax.fori_loop(0, n_chunks, step, 0)
        plsc.subcore_barrier()
        pltpu.sync_copy(acc.at[pl.ds(t * _STRIPE, _STRIPE)],
                        out.at[pl.ds(c * _NPAD + t * _STRIPE, _STRIPE)])

    return pl.kernel(
        body,
        out_type=jax.ShapeDtypeStruct((_NC * _NPAD,), jnp.float32),
        mesh=_sc_mesh(),
        scratch_types=[
            pltpu.VMEM((1, _E // _NW), jnp.int32),
            pltpu.VMEM((_CHUNK,), jnp.int32),
            pltpu.VMEM((_CHUNK,), jnp.float32),
            pltpu.VMEM_SHARED((_NPAD,), jnp.float32),
        ],
    )


def _make_sc_agg(edge_split):
    if edge_split:
        ept = _E // _NC // _NS
    else:
        ept = _E // _NS
    n_groups = ept // 16
    n_chunks4 = ept // _CHUNK

    def body(src3, dst3, table, out,
             sidx, didx2, s1a, d1a, s1b, d1b, s1c, d1c, s1d, d1d,
             rows0, rows1, rows2, rows3, zbuf, sem0, sem1, sem2, sem3, acc):
        c = lax.axis_index("c")
        t = lax.axis_index("s")
        if edge_split:
            w = c * _NS + t
        else:
            w = t

        def fillz(i, carry):
            r = i // 8
            k = i % 8
            zbuf[r, pl.ds(k * 16, 16)] = jnp.zeros((16,), jnp.float32)
            return carry

        lax.fori_loop(0, (_AZ // 4) * 8, fillz, 0)
        pltpu.sync_copy(src3.at[w], sidx)
        pltpu.sync_copy(dst3.at[w], didx2)
        if not edge_split:
            off = c * _N

            def addoff(g, carry):
                v = sidx[0, pl.ds(g * 16, 16)]
                sidx[0, pl.ds(g * 16, 16)] = v + off
                return carry

            lax.fori_loop(0, n_groups, addoff, 0)

        def copy80(src1d, base, dst80):
            for k in range(5):
                dst80[pl.ds(k * 16, 16)] = src1d[0, pl.ds(base + k * 16, 16)]

        def g_start(idx80, rows, sem):
            pltpu.async_copy(table.at[idx80], rows, sem)

        def g_wait(rows, sem):
            pltpu.make_async_copy(table.at[pl.ds(0, _CHUNK)], rows,
                                  sem).wait()

        def one_pass(p, dst3_again):
            lo = p * _WIN
            for q in range(4):
                pltpu.sync_copy(
                    zbuf, acc.at[pl.ds(t * _AZ + q * (_AZ // 4), _AZ // 4)])

            pltpu.sync_copy(dst3_again.at[w], didx2)

            def remap(g, carry):
                v = didx2[0, pl.ds(g * 16, 16)]
                ok = (v >= lo) & (v < lo + _WIN)
                junk = _WIN + (v & 0x7F)
                didx2[0, pl.ds(g * 16, 16)] = jnp.where(ok, v - lo, junk)
                return carry

            lax.fori_loop(0, n_groups, remap, 0)
            plsc.subcore_barrier()

            bufs = ((s1a, d1a, rows0, sem0), (s1b, d1b, rows1, sem1),
                    (s1c, d1c, rows2, sem2), (s1d, d1d, rows3, sem3))

            def cp_start(ch, b):
                sb, db, rb, sm = bufs[b]
                copy80(sidx, ch * _CHUNK, sb)
                copy80(didx2, ch * _CHUNK, db)
                pltpu.async_copy(table.at[sb], rb, sm)

            def wait_scat(b):
                sb, db, rb, sm = bufs[b]
                g_wait(rb, sm)
                pltpu.sync_copy(rb, acc.at[db], add=True)

            quads = n_chunks4 // 4
            rem = n_chunks4 % 4
            cp_start(0, 0)
            cp_start(1, 1)

            def quad(i, carry):
                c0 = 4 * i
                cp_start(c0 + 2, 2)
                cp_start(c0 + 3, 3)
                wait_scat(0)
                wait_scat(1)
                cp_start(c0 + 4, 0)
                cp_start(c0 + 5, 1)
                wait_scat(2)
                wait_scat(3)
                return carry

            lax.fori_loop(0, quads - 1, quad, 0)
            cbase = 4 * (quads - 1)
            cp_start(cbase + 2, 2)
            cp_start(cbase + 3, 3)
            wait_scat(0)
            wait_scat(1)
            if rem >= 1:
                cp_start(4 * quads, 0)
            if rem == 2:
                cp_start(4 * quads + 1, 1)
            wait_scat(2)
            wait_scat(3)
            if rem >= 1:
                wait_scat(0)
            if rem == 2:
                wait_scat(1)
            plsc.subcore_barrier()
            obase = c * _NPAD2 + p * _WIN + t * _AW
            pltpu.sync_copy(acc.at[pl.ds(t * _AW, _AW)],
                            out.at[pl.ds(obase, _AW)])
            plsc.subcore_barrier()

        def passes(p, carry):
            one_pass(p, dst3)
            return carry

        for p in range(_NWIN):
            one_pass(p, dst3)

    return pl.kernel(
        body,
        out_type=jax.ShapeDtypeStruct((_NC * _NPAD2, 128), jnp.float32),
        mesh=_sc_mesh(),
        scratch_types=[
            pltpu.VMEM((1, ept), jnp.int32),
            pltpu.VMEM((1, ept), jnp.int32),
            pltpu.VMEM((_CHUNK,), jnp.int32),
            pltpu.VMEM((_CHUNK,), jnp.int32),
            pltpu.VMEM((_CHUNK,), jnp.int32),
            pltpu.VMEM((_CHUNK,), jnp.int32),
            pltpu.VMEM((_CHUNK,), jnp.int32),
            pltpu.VMEM((_CHUNK,), jnp.int32),
            pltpu.VMEM((_CHUNK,), jnp.int32),
            pltpu.VMEM((_CHUNK,), jnp.int32),
            pltpu.VMEM((_CHUNK, 128), jnp.float32),
            pltpu.VMEM((_CHUNK, 128), jnp.float32),
            pltpu.VMEM((_CHUNK, 128), jnp.float32),
            pltpu.VMEM((_CHUNK, 128), jnp.float32),
            pltpu.VMEM((_AZ // 4, 128), jnp.float32),
            pltpu.SemaphoreType.DMA,
            pltpu.SemaphoreType.DMA,
            pltpu.SemaphoreType.DMA,
            pltpu.SemaphoreType.DMA,
            pltpu.VMEM_SHARED((_ACC, 128), jnp.float32),
        ],
    )


_BN = 1000


def _tc_deginv(degp):
    blk = 2048

    def body(d_ref, out_ref):
        deg = d_ref[0, :] + d_ref[1, :] + 1.0
        out_ref[...] = lax.rsqrt(deg)[:, None]

    return pl.pallas_call(
        body,
        grid=(_NPAD // blk,),
        in_specs=[pl.BlockSpec((2, blk), lambda i: (0, i))],
        out_specs=pl.BlockSpec((blk, 1), lambda i: (i, 0)),
        out_shape=jax.ShapeDtypeStruct((_NPAD, 1), jnp.float32),
    )(degp)


def _tc_stage1(dinv, x, W1s):
    def body(dv_ref, x_ref, w_ref, out_ref):
        h = jnp.dot(x_ref[...], w_ref[0],
                    preferred_element_type=jnp.float32)
        out_ref[0] = dv_ref[...] * h

    return pl.pallas_call(
        body,
        grid=(_N // _BN, 2),
        in_specs=[
            pl.BlockSpec((_BN, 1), lambda i, c: (i, 0)),
            pl.BlockSpec((_BN, 128), lambda i, c: (i, 0)),
            pl.BlockSpec((1, 128, 128), lambda i, c: (c, 0, 0)),
        ],
        out_specs=pl.BlockSpec((1, _BN, 128), lambda i, c: (c, i, 0)),
        out_shape=jax.ShapeDtypeStruct((2, _N, 128), jnp.float32),
    )(dinv, x, W1s)


def _tc_stage2(dinv, s1, g1, W2, b1r):
    def body(dv_ref, s_ref, g_ref, w_ref, b_ref, out_ref):
        dv = dv_ref[...]
        b = b_ref[...]
        w = w_ref[...]
        z0 = jnp.maximum(dv * (s_ref[0] + g_ref[0]) + b[0, :128], 0.0)
        z1 = jnp.maximum(dv * (s_ref[1] + g_ref[1]) + b[0, 128:], 0.0)
        h = (jnp.dot(z0, w[:128], preferred_element_type=jnp.float32)
             + jnp.dot(z1, w[128:], preferred_element_type=jnp.float32))
        out_ref[...] = dv * h

    return pl.pallas_call(
        body,
        grid=(_N // _BN,),
        in_specs=[
            pl.BlockSpec((_BN, 1), lambda i: (i, 0)),
            pl.BlockSpec((2, _BN, 128), lambda i: (0, i, 0)),
            pl.BlockSpec((2, _BN, 128), lambda i: (0, i, 0)),
            pl.BlockSpec((256, 128), lambda i: (0, 0)),
            pl.BlockSpec((1, 256), lambda i: (0, 0)),
        ],
        out_specs=pl.BlockSpec((_BN, 128), lambda i: (i, 0)),
        out_shape=jax.ShapeDtypeStruct((_N, 128), jnp.float32),
    )(dinv, s1, g1, W2, b1r)


def _tc_stage3(dinv, s2, g2, b2r):
    def body(dv_ref, s_ref, g_ref, b_ref, out_ref):
        tot = s_ref[0] + s_ref[1] + g_ref[...]
        out_ref[...] = jnp.maximum(dv_ref[...] * tot + b_ref[...], 0.0)

    return pl.pallas_call(
        body,
        grid=(_N // _BN,),
        in_specs=[
            pl.BlockSpec((_BN, 1), lambda i: (i, 0)),
            pl.BlockSpec((2, _BN, 128), lambda i: (0, i, 0)),
            pl.BlockSpec((_BN, 128), lambda i: (i, 0)),
            pl.BlockSpec((1, 128), lambda i: (0, 0)),
        ],
        out_specs=pl.BlockSpec((_BN, 128), lambda i: (i, 0)),
        out_shape=jax.ShapeDtypeStruct((_N, 128), jnp.float32),
    )(dinv, s2, g2, b2r)


def kernel(x, edge_index, W1, b1, W2, b2):
    src_all = edge_index[0].reshape(_NS, 1, _E // _NS)
    dst_all = edge_index[1].reshape(_NS, 1, _E // _NS)
    src_half = edge_index[0].reshape(_NW, 1, _E // _NW)
    dst_half = edge_index[1].reshape(_NW, 1, _E // _NW)
    deg_dst = edge_index[1].reshape(_NW, 1, _E // _NW)
    zeros1 = jnp.zeros((_STRIPE,), jnp.float32)
    ones1 = jnp.ones((_CHUNK,), jnp.float32)
    W1s = W1.reshape(128, 2, 128).transpose(1, 0, 2)

    degp = _make_sc_degree()(deg_dst, ones1, zeros1)
    dinv = _tc_deginv(degp.reshape(_NC, _NPAD))

    g1 = _tc_stage1(dinv, x, W1s)
    s1 = _make_sc_agg(edge_split=False)(
        src_all, dst_all, g1.reshape(2 * _N, 128))
    g2 = _tc_stage2(dinv, s1.reshape(_NC, _NPAD2, 128), g1, W2,
                    b1.reshape(1, 256))
    s2 = _make_sc_agg(edge_split=True)(
        src_half, dst_half, g2)
    return _tc_stage3(dinv, s2.reshape(_NC, _NPAD2, 128), g2,
                      b2.reshape(1, 128))


# speedup vs baseline: 11.4945x; 1.0018x over previous
import jax
import jax.numpy as jnp
from jax import lax
from jax.experimental import pallas as pl
from jax.experimental.pallas import tpu as pltpu
from jax.experimental.pallas import tpu_sc as plsc

_N = 10000
_E = 320000
_NPAD = 10240
_NC = 2
_NS = 16
_NW = _NC * _NS
_CHUNK = 80
_STRIPE = _NPAD // _NS
_WIN = 3456
_NWIN = 3
_NPAD2 = _NWIN * _WIN
_ACC = 3584
_AZ = _ACC // _NS
_AW = _WIN // _NS


def _sc_mesh():
    return plsc.VectorSubcoreMesh(core_axis_name="c", subcore_axis_name="s")


def _make_sc_degree():
    n_chunks = _E // _NW // _CHUNK

    def body(dst3, ones_hbm, zinit, out, didx, didx1, ones_v, acc):
        c = lax.axis_index("c")
        t = lax.axis_index("s")
        w = c * _NS + t
        pltpu.sync_copy(zinit, acc.at[pl.ds(t * _STRIPE, _STRIPE)])
        pltpu.sync_copy(dst3.at[w], didx)
        pltpu.sync_copy(ones_hbm, ones_v)
        plsc.subcore_barrier()

        def step(j, carry):
            base = j * _CHUNK
            for k in range(5):
                didx1[pl.ds(k * 16, 16)] = didx[0, pl.ds(base + k * 16, 16)]
            pltpu.sync_copy(ones_v, acc.at[didx1], add=True)
            return carry

        lax.fori_loop(0, n_chunks, step, 0)
        plsc.subcore_barrier()
        pltpu.sync_copy(acc.at[pl.ds(t * _STRIPE, _STRIPE)],
                        out.at[pl.ds(c * _NPAD + t * _STRIPE, _STRIPE)])

    return pl.kernel(
        body,
        out_type=jax.ShapeDtypeStruct((_NC * _NPAD,), jnp.float32),
        mesh=_sc_mesh(),
        scratch_types=[
            pltpu.VMEM((1, _E // _NW), jnp.int32),
            pltpu.VMEM((_CHUNK,), jnp.int32),
            pltpu.VMEM((_CHUNK,), jnp.float32),
            pltpu.VMEM_SHARED((_NPAD,), jnp.float32),
        ],
    )


def _make_sc_agg(edge_split):
    if edge_split:
        ept = _E // _NC // _NS
    else:
        ept = _E // _NS
    n_groups = ept // 16
    n_chunks4 = ept // _CHUNK

    def body(src3, dst3, table, out,
             sidx, didx2, s1a, d1a, s1b, d1b, s1c, d1c, s1d, d1d,
             rows0, rows1, rows2, rows3, zbuf, sem0, sem1, sem2, sem3, acc):
        c = lax.axis_index("c")
        t = lax.axis_index("s")
        if edge_split:
            w = c * _NS + t
        else:
            w = t

        def fillz(i, carry):
            r = i // 8
            k = i % 8
            zbuf[r, pl.ds(k * 16, 16)] = jnp.zeros((16,), jnp.float32)
            return carry

        lax.fori_loop(0, (_AZ // 4) * 8, fillz, 0)
        pltpu.sync_copy(src3.at[w], sidx)
        pltpu.sync_copy(dst3.at[w], didx2)
        if not edge_split:
            off = c * _N

            def addoff(g, carry):
                v = sidx[0, pl.ds(g * 16, 16)]
                sidx[0, pl.ds(g * 16, 16)] = v + off
                return carry

            lax.fori_loop(0, n_groups, addoff, 0)

        def copy80(src1d, base, dst80):
            for k in range(5):
                dst80[pl.ds(k * 16, 16)] = src1d[0, pl.ds(base + k * 16, 16)]

        def g_start(idx80, rows, sem):
            pltpu.async_copy(table.at[idx80], rows, sem)

        def g_wait(rows, sem):
            pltpu.make_async_copy(table.at[pl.ds(0, _CHUNK)], rows,
                                  sem).wait()

        def one_pass(p, dst3_again):
            lo = p * _WIN
            for q in range(4):
                pltpu.sync_copy(
                    zbuf, acc.at[pl.ds(t * _AZ + q * (_AZ // 4), _AZ // 4)])

            pltpu.sync_copy(dst3_again.at[w], didx2)

            def remap(g, carry):
                v = didx2[0, pl.ds(g * 16, 16)]
                ok = (v >= lo) & (v < lo + _WIN)
                junk = _WIN + (v & 0x7F)
                didx2[0, pl.ds(g * 16, 16)] = jnp.where(ok, v - lo, junk)
                return carry

            lax.fori_loop(0, n_groups, remap, 0)
            plsc.subcore_barrier()

            bufs = ((s1a, d1a, rows0, sem0), (s1b, d1b, rows1, sem1),
                    (s1c, d1c, rows2, sem2), (s1d, d1d, rows3, sem3))

            def cp_start(ch, b):
                sb, db, rb, sm = bufs[b]
                copy80(sidx, ch * _CHUNK, sb)
                copy80(didx2, ch * _CHUNK, db)
                pltpu.async_copy(table.at[sb], rb, sm)

            def wait_scat(b):
                sb, db, rb, sm = bufs[b]
                g_wait(rb, sm)
                pltpu.sync_copy(rb, acc.at[db], add=True)

            quads = n_chunks4 // 4
            rem = n_chunks4 % 4
            cp_start(0, 0)
            cp_start(1, 1)

            def quad(i, carry):
                c0 = 4 * i
                cp_start(c0 + 2, 2)
                cp_start(c0 + 3, 3)
                wait_scat(0)
                wait_scat(1)
                cp_start(c0 + 4, 0)
                cp_start(c0 + 5, 1)
                wait_scat(2)
                wait_scat(3)
                return carry

            lax.fori_loop(0, quads - 1, quad, 0)
            cbase = 4 * (quads - 1)
            cp_start(cbase + 2, 2)
            cp_start(cbase + 3, 3)
            wait_scat(0)
            wait_scat(1)
            if rem >= 1:
                cp_start(4 * quads, 0)
            if rem == 2:
                cp_start(4 * quads + 1, 1)
            wait_scat(2)
            wait_scat(3)
            if rem >= 1:
                wait_scat(0)
            if rem == 2:
                wait_scat(1)
            plsc.subcore_barrier()
            obase = c * _NPAD2 + p * _WIN + t * _AW
            pltpu.sync_copy(acc.at[pl.ds(t * _AW, _AW)],
                            out.at[pl.ds(obase, _AW)])
            plsc.subcore_barrier()

        for p in range(_NWIN):
            one_pass(p, dst3)

    return pl.kernel(
        body,
        out_type=jax.ShapeDtypeStruct((_NC * _NPAD2, 128), jnp.float32),
        mesh=_sc_mesh(),
        scratch_types=[
            pltpu.VMEM((1, ept), jnp.int32),
            pltpu.VMEM((1, ept), jnp.int32),
            pltpu.VMEM((_CHUNK,), jnp.int32),
            pltpu.VMEM((_CHUNK,), jnp.int32),
            pltpu.VMEM((_CHUNK,), jnp.int32),
            pltpu.VMEM((_CHUNK,), jnp.int32),
            pltpu.VMEM((_CHUNK,), jnp.int32),
            pltpu.VMEM((_CHUNK,), jnp.int32),
            pltpu.VMEM((_CHUNK,), jnp.int32),
            pltpu.VMEM((_CHUNK,), jnp.int32),
            pltpu.VMEM((_CHUNK, 128), jnp.float32),
            pltpu.VMEM((_CHUNK, 128), jnp.float32),
            pltpu.VMEM((_CHUNK, 128), jnp.float32),
            pltpu.VMEM((_CHUNK, 128), jnp.float32),
            pltpu.VMEM((_AZ // 4, 128), jnp.float32),
            pltpu.SemaphoreType.DMA,
            pltpu.SemaphoreType.DMA,
            pltpu.SemaphoreType.DMA,
            pltpu.SemaphoreType.DMA,
            pltpu.VMEM_SHARED((_ACC, 128), jnp.float32),
        ],
    )


_BN = 1000


def _tc_deginv(degp):
    blk = 2048

    def body(d_ref, out_ref):
        deg = d_ref[0, :] + d_ref[1, :] + 1.0
        out_ref[...] = lax.rsqrt(deg)[:, None]

    return pl.pallas_call(
        body,
        grid=(_NPAD // blk,),
        in_specs=[pl.BlockSpec((2, blk), lambda i: (0, i))],
        out_specs=pl.BlockSpec((blk, 1), lambda i: (i, 0)),
        out_shape=jax.ShapeDtypeStruct((_NPAD, 1), jnp.float32),
    )(degp)


def _tc_stage1(dinv, x, W1s):
    def body(dv_ref, x_ref, w_ref, out_ref):
        h = jnp.dot(x_ref[...], w_ref[0],
                    preferred_element_type=jnp.float32)
        out_ref[0] = dv_ref[...] * h

    return pl.pallas_call(
        body,
        grid=(_N // _BN, 2),
        in_specs=[
            pl.BlockSpec((_BN, 1), lambda i, c: (i, 0)),
            pl.BlockSpec((_BN, 128), lambda i, c: (i, 0)),
            pl.BlockSpec((1, 128, 128), lambda i, c: (c, 0, 0)),
        ],
        out_specs=pl.BlockSpec((1, _BN, 128), lambda i, c: (c, i, 0)),
        out_shape=jax.ShapeDtypeStruct((2, _N, 128), jnp.float32),
    )(dinv, x, W1s)


def _tc_stage2(dinv, s1, g1, W2, b1r):
    def body(dv_ref, s_ref, g_ref, w_ref, b_ref, out_ref):
        dv = dv_ref[...]
        b = b_ref[...]
        w = w_ref[...]
        z0 = jnp.maximum(dv * (s_ref[0] + g_ref[0]) + b[0, :128], 0.0)
        z1 = jnp.maximum(dv * (s_ref[1] + g_ref[1]) + b[0, 128:], 0.0)
        h = (jnp.dot(z0, w[:128], preferred_element_type=jnp.float32)
             + jnp.dot(z1, w[128:], preferred_element_type=jnp.float32))
        out_ref[...] = dv * h

    return pl.pallas_call(
        body,
        grid=(_N // _BN,),
        in_specs=[
            pl.BlockSpec((_BN, 1), lambda i: (i, 0)),
            pl.BlockSpec((2, _BN, 128), lambda i: (0, i, 0)),
            pl.BlockSpec((2, _BN, 128), lambda i: (0, i, 0)),
            pl.BlockSpec((256, 128), lambda i: (0, 0)),
            pl.BlockSpec((1, 256), lambda i: (0, 0)),
        ],
        out_specs=pl.BlockSpec((_BN, 128), lambda i: (i, 0)),
        out_shape=jax.ShapeDtypeStruct((_N, 128), jnp.float32),
    )(dinv, s1, g1, W2, b1r)


def _tc_stage3(dinv, s2, g2, b2r):
    def body(dv_ref, s_ref, g_ref, b_ref, out_ref):
        tot = s_ref[0] + s_ref[1] + g_ref[...]
        out_ref[...] = jnp.maximum(dv_ref[...] * tot + b_ref[...], 0.0)

    return pl.pallas_call(
        body,
        grid=(_N // _BN,),
        in_specs=[
            pl.BlockSpec((_BN, 1), lambda i: (i, 0)),
            pl.BlockSpec((2, _BN, 128), lambda i: (0, i, 0)),
            pl.BlockSpec((_BN, 128), lambda i: (i, 0)),
            pl.BlockSpec((1, 128), lambda i: (0, 0)),
        ],
        out_specs=pl.BlockSpec((_BN, 128), lambda i: (i, 0)),
        out_shape=jax.ShapeDtypeStruct((_N, 128), jnp.float32),
    )(dinv, s2, g2, b2r)


def kernel(x, edge_index, W1, b1, W2, b2):
    src_all = edge_index[0].reshape(_NS, 1, _E // _NS)
    dst_all = edge_index[1].reshape(_NS, 1, _E // _NS)
    src_half = edge_index[0].reshape(_NW, 1, _E // _NW)
    dst_half = edge_index[1].reshape(_NW, 1, _E // _NW)
    deg_dst = edge_index[1].reshape(_NW, 1, _E // _NW)
    zeros1 = jnp.zeros((_STRIPE,), jnp.float32)
    ones1 = jnp.ones((_CHUNK,), jnp.float32)
    W1s = W1.reshape(128, 2, 128).transpose(1, 0, 2)

    degp = _make_sc_degree()(deg_dst, ones1, zeros1)
    dinv = _tc_deginv(degp.reshape(_NC, _NPAD))

    g1 = _tc_stage1(dinv, x, W1s)
    s1 = _make_sc_agg(edge_split=False)(
        src_all, dst_all, g1.reshape(2 * _N, 128))
    g2 = _tc_stage2(dinv, s1.reshape(_NC, _NPAD2, 128), g1, W2,
                    b1.reshape(1, 256))
    s2 = _make_sc_agg(edge_split=True)(
        src_half, dst_half, g2)
    return _tc_stage3(dinv, s2.reshape(_NC, _NPAD2, 128), g2,
                      b2.reshape(1, 128))
